# baseline (speedup 1.0000x reference)
import functools

import jax
import jax.numpy as jnp
from jax import lax
from jax.experimental import pallas as pl
from jax.experimental.pallas import tpu as pltpu
from jax.experimental.pallas import tpu_sc as plsc

N = 10000
E = 320000
CH = 128
HALF = 64
DW = 8

NC = 2
NS = 16

CHUNK = 64
CPT = 160
NCHUNK = NC * NS * CPT
EPAD = NCHUNK * CHUNK

DCHUNK = 128
DNCHUNK = EPAD // DCHUNK
DCPT = DNCHUNK // (NC * NS)

ACCROWS = 10240
RPT = ACCROWS // NS
LASTM = N - (NS - 1) * RPT


def _sc_deg_body(dstc, zeros8, ones8, out, acc, dst_v, ones_v):
    c = lax.axis_index("c")
    s = lax.axis_index("s")
    r0 = s * RPT
    pltpu.sync_copy(zeros8.at[pl.ds(r0, RPT)], acc.at[pl.ds(r0, RPT)])
    pltpu.sync_copy(ones8, ones_v)
    base = c * (DNCHUNK // NC) + s * DCPT
    pltpu.sync_copy(dstc.at[pl.ds(base, DCPT)], dst_v)
    plsc.subcore_barrier()

    def body(k, carry):
        pltpu.sync_copy(ones_v, acc.at[dst_v.at[k]], add=True)
        return carry

    lax.fori_loop(0, DCPT, body, 0)
    plsc.subcore_barrier()
    pltpu.sync_copy(acc.at[pl.ds(r0, RPT)], out.at[c].at[pl.ds(r0, RPT)])


def _sc_conv_body(m, edgec, zeros, out, acc, idx_v, bufs, sem0, sem1):
    c = lax.axis_index("c")
    s = lax.axis_index("s")
    r0 = s * RPT

    @pl.when(jnp.logical_and(c == 0, s < NS - 1))
    def _():
        pltpu.sync_copy(m.at[pl.ds(r0, RPT)], acc.at[pl.ds(r0, RPT)])

    @pl.when(jnp.logical_and(c == 0, s == NS - 1))
    def _():
        pltpu.sync_copy(m.at[pl.ds((NS - 1) * RPT, LASTM)],
                        acc.at[pl.ds((NS - 1) * RPT, LASTM)])
        pltpu.sync_copy(zeros.at[pl.ds(N, ACCROWS - N)],
                        acc.at[pl.ds(N, ACCROWS - N)])

    @pl.when(c == 1)
    def _():
        pltpu.sync_copy(zeros.at[pl.ds(r0, RPT)], acc.at[pl.ds(r0, RPT)])

    base = c * (NCHUNK // NC) + s * CPT
    pltpu.sync_copy(edgec.at[pl.ds(base, CPT)], idx_v)
    plsc.subcore_barrier()

    sems = (sem0, sem1)

    def start(g, par):
        pltpu.async_copy(m.at[idx_v.at[g, pl.ds(0, CHUNK)]], bufs.at[par],
                         sems[par])

    def drain_scatter(g, par):
        pltpu.make_async_copy(m.at[pl.ds(0, CHUNK)], bufs.at[par],
                              sems[par]).wait()
        pltpu.sync_copy(bufs.at[par], acc.at[idx_v.at[g, pl.ds(CHUNK, CHUNK)]],
                        add=True)

    start(0, 0)
    start(1, 1)

    def body(gg, carry):
        g0 = gg * 2
        for par in (0, 1):
            g = g0 + par
            drain_scatter(g, par)

            @pl.when(g + 2 < CPT)
            def _():
                start(g + 2, par)

        return carry

    lax.fori_loop(0, CPT // 2, body, 0)
    plsc.subcore_barrier()
    pltpu.sync_copy(acc.at[pl.ds(r0, RPT)], out.at[c].at[pl.ds(r0, RPT)])


def _dis_from_degp(degp_ref):
    deg = degp_ref[0, :, 0:1] + degp_ref[1, :, 0:1] + 1.0
    return lax.rsqrt(deg)[:N, :]


def _tc_m1_body(x_ref, w1_ref, degp_ref, out_ref):
    dis = _dis_from_degp(degp_ref)
    h = jnp.dot(x_ref[...], w1_ref[...], preferred_element_type=jnp.float32)
    out_ref[...] = h * dis


def _tc_m2_body(p_ref, degp_ref, b1_ref, wcat_ref, out_ref):
    dis = _dis_from_degp(degp_ref)
    a = p_ref[0, :N, :] + p_ref[1, :N, :]
    h = jnp.maximum(dis * a + b1_ref[...][None, :], 0.0)
    out_ref[...] = jnp.dot(h, wcat_ref[...],
                           preferred_element_type=jnp.float32) * dis


def _tc_out_body(q_ref, degp_ref, bmu_ref, bls_ref, mu_ref, ls_ref):
    dis = _dis_from_degp(degp_ref)
    o = dis * (q_ref[0, :N, :] + q_ref[1, :N, :])
    mu_ref[...] = o[:, :HALF] + bmu_ref[...][None, :]
    ls_ref[...] = o[:, HALF:] + bls_ref[...][None, :]


def _make_sc_kernels():
    mesh = plsc.VectorSubcoreMesh(core_axis_name="c", subcore_axis_name="s")
    deg_kernel = functools.partial(
        pl.kernel,
        out_type=jax.ShapeDtypeStruct((NC, ACCROWS, DW), jnp.float32),
        mesh=mesh,
        compiler_params=pltpu.CompilerParams(use_tc_tiling_on_sc=False),
        scratch_types=[
            pltpu.VMEM_SHARED((ACCROWS, DW), jnp.float32),
            pltpu.VMEM((DCPT, DCHUNK), jnp.int32),
            pltpu.VMEM((DCHUNK, DW), jnp.float32),
        ],
    )(_sc_deg_body)
    conv_kernel = functools.partial(
        pl.kernel,
        out_type=jax.ShapeDtypeStruct((NC, ACCROWS, CH), jnp.float32),
        mesh=mesh,
        scratch_types=[
            pltpu.VMEM_SHARED((ACCROWS, CH), jnp.float32),
            pltpu.VMEM((CPT, 2 * CHUNK), jnp.int32),
            pltpu.VMEM((2, CHUNK, CH), jnp.float32),
            pltpu.SemaphoreType.DMA,
            pltpu.SemaphoreType.DMA,
        ],
    )(_sc_conv_body)
    return deg_kernel, conv_kernel


def kernel(x, edge_index, W1, b1, Wmu, bmu, Wls, bls):
    src = edge_index[0].astype(jnp.int32)
    dst = edge_index[1].astype(jnp.int32)
    npad = EPAD - E
    ar = jnp.arange(npad, dtype=jnp.int32)
    src_p = jnp.concatenate([src, ar % N])
    dst_p = jnp.concatenate([dst, N + ar % (ACCROWS - N)])
    edgec = jnp.concatenate(
        [src_p.reshape(NCHUNK, CHUNK), dst_p.reshape(NCHUNK, CHUNK)], axis=1)
    dstc_deg = dst_p.reshape(DNCHUNK, DCHUNK)
    zeros128 = jnp.zeros((ACCROWS, CH), jnp.float32)
    zeros8 = jnp.zeros((ACCROWS, DW), jnp.float32)
    ones8 = jnp.ones((DCHUNK, DW), jnp.float32)

    deg_kernel, conv_kernel = _make_sc_kernels()

    degp = deg_kernel(dstc_deg, zeros8, ones8)

    m1 = pl.pallas_call(
        _tc_m1_body,
        out_shape=jax.ShapeDtypeStruct((N, CH), jnp.float32),
    )(x, W1, degp)

    p1 = conv_kernel(m1, edgec, zeros128)

    wcat = jnp.concatenate([Wmu, Wls], axis=1)
    m2 = pl.pallas_call(
        _tc_m2_body,
        out_shape=jax.ShapeDtypeStruct((N, CH), jnp.float32),
    )(p1, degp, b1, wcat)

    p2 = conv_kernel(m2, edgec, zeros128)

    mu, logstd = pl.pallas_call(
        _tc_out_body,
        out_shape=(jax.ShapeDtypeStruct((N, HALF), jnp.float32),
                   jax.ShapeDtypeStruct((N, HALF), jnp.float32)),
    )(p2, degp, bmu, bls)
    return (mu, logstd)


# speedup vs baseline: 1.1564x; 1.1564x over previous
import functools

import jax
import jax.numpy as jnp
from jax import lax
from jax.experimental import pallas as pl
from jax.experimental.pallas import tpu as pltpu
from jax.experimental.pallas import tpu_sc as plsc

N = 10000
E = 320000
CH = 128
HALF = 64
DW = 8

NC = 2
NS = 16

CHUNK = 64
CPT = 160
NCHUNK = NC * NS * CPT
EPAD = NCHUNK * CHUNK

DCHUNK = 128
DNCHUNK = EPAD // DCHUNK
DCPT = DNCHUNK // (NC * NS)

ACCROWS = 10240
RPT = ACCROWS // NS
LASTM = N - (NS - 1) * RPT


def _sc_deg_body(dstc, zeros8, ones8, out, acc, dst_v, ones_v, dsem):
    c = lax.axis_index("c")
    s = lax.axis_index("s")
    r0 = s * RPT
    pltpu.sync_copy(zeros8.at[pl.ds(r0, RPT)], acc.at[pl.ds(r0, RPT)])
    pltpu.sync_copy(ones8, ones_v)
    base = c * (DNCHUNK // NC) + s * DCPT
    pltpu.sync_copy(dstc.at[pl.ds(base, DCPT)], dst_v)
    plsc.subcore_barrier()

    def body(k, carry):
        pltpu.async_copy(ones_v, acc.at[dst_v.at[k]], dsem, add=True)

        @pl.when(k >= 8)
        def _():
            pltpu.make_async_copy(ones_v, acc.at[pl.ds(0, DCHUNK)],
                                  dsem).wait()

        return carry

    lax.fori_loop(0, DCPT, body, 0)
    for _ in range(8):
        pltpu.make_async_copy(ones_v, acc.at[pl.ds(0, DCHUNK)], dsem).wait()
    plsc.subcore_barrier()
    pltpu.sync_copy(acc.at[pl.ds(r0, RPT)], out.at[c].at[pl.ds(r0, RPT)])


def _sc_conv_body(m, edgec, zeros, out, acc, idx_v, bufs,
                  gsem0, gsem1, gsem2, ssem0, ssem1, ssem2):
    c = lax.axis_index("c")
    s = lax.axis_index("s")
    r0 = s * RPT

    @pl.when(jnp.logical_and(c == 0, s < NS - 1))
    def _():
        pltpu.sync_copy(m.at[pl.ds(r0, RPT)], acc.at[pl.ds(r0, RPT)])

    @pl.when(jnp.logical_and(c == 0, s == NS - 1))
    def _():
        pltpu.sync_copy(m.at[pl.ds((NS - 1) * RPT, LASTM)],
                        acc.at[pl.ds((NS - 1) * RPT, LASTM)])
        pltpu.sync_copy(zeros.at[pl.ds(N, ACCROWS - N)],
                        acc.at[pl.ds(N, ACCROWS - N)])

    @pl.when(c == 1)
    def _():
        pltpu.sync_copy(zeros.at[pl.ds(r0, RPT)], acc.at[pl.ds(r0, RPT)])

    base = c * (NCHUNK // NC) + s * CPT
    pltpu.sync_copy(edgec.at[pl.ds(base, CPT)], idx_v)
    plsc.subcore_barrier()

    gsems = (gsem0, gsem1, gsem2)
    ssems = (ssem0, ssem1, ssem2)

    def start_gather(g, r):
        pltpu.async_copy(m.at[idx_v.at[g, pl.ds(0, CHUNK)]], bufs.at[r],
                         gsems[r])

    def wait_gather(r):
        pltpu.make_async_copy(m.at[pl.ds(0, CHUNK)], bufs.at[r],
                              gsems[r]).wait()

    def start_scatter(g, r):
        pltpu.async_copy(bufs.at[r], acc.at[idx_v.at[g, pl.ds(CHUNK, CHUNK)]],
                         ssems[r], add=True)

    def wait_scatter(r):
        pltpu.make_async_copy(bufs.at[r], acc.at[pl.ds(0, CHUNK)],
                              ssems[r]).wait()

    start_gather(0, 0)
    start_gather(1, 1)

    def body(gg, carry):
        g0 = gg * 3
        for r in (0, 1, 2):
            g = g0 + r
            r2 = (r + 2) % 3
            wait_gather(r)
            start_scatter(g, r)

            @pl.when(g + 2 < CPT)
            def _():
                @pl.when(g >= 1)
                def _():
                    wait_scatter(r2)

                start_gather(g + 2, r2)

        return carry

    lax.fori_loop(0, CPT // 3, body, 0)
    wait_gather(0)
    start_scatter(CPT - 1, 0)
    wait_scatter(2)
    wait_scatter(1)
    wait_scatter(0)
    plsc.subcore_barrier()
    pltpu.sync_copy(acc.at[pl.ds(r0, RPT)], out.at[c].at[pl.ds(r0, RPT)])


def _dis_from_degp(degp_ref):
    deg = degp_ref[0, :, 0:1] + degp_ref[1, :, 0:1] + 1.0
    return lax.rsqrt(deg)[:N, :]


def _tc_m1_body(x_ref, w1_ref, degp_ref, out_ref):
    dis = _dis_from_degp(degp_ref)
    h = jnp.dot(x_ref[...], w1_ref[...], preferred_element_type=jnp.float32)
    out_ref[...] = h * dis


def _tc_m2_body(p_ref, degp_ref, b1_ref, wcat_ref, out_ref):
    dis = _dis_from_degp(degp_ref)
    a = p_ref[0, :N, :] + p_ref[1, :N, :]
    h = jnp.maximum(dis * a + b1_ref[...][None, :], 0.0)
    out_ref[...] = jnp.dot(h, wcat_ref[...],
                           preferred_element_type=jnp.float32) * dis


def _tc_out_body(q_ref, degp_ref, bmu_ref, bls_ref, mu_ref, ls_ref):
    dis = _dis_from_degp(degp_ref)
    o = dis * (q_ref[0, :N, :] + q_ref[1, :N, :])
    mu_ref[...] = o[:, :HALF] + bmu_ref[...][None, :]
    ls_ref[...] = o[:, HALF:] + bls_ref[...][None, :]


def _make_sc_kernels():
    mesh = plsc.VectorSubcoreMesh(core_axis_name="c", subcore_axis_name="s")
    deg_kernel = functools.partial(
        pl.kernel,
        out_type=jax.ShapeDtypeStruct((NC, ACCROWS, DW), jnp.float32),
        mesh=mesh,
        compiler_params=pltpu.CompilerParams(use_tc_tiling_on_sc=False),
        scratch_types=[
            pltpu.VMEM_SHARED((ACCROWS, DW), jnp.float32),
            pltpu.VMEM((DCPT, DCHUNK), jnp.int32),
            pltpu.VMEM((DCHUNK, DW), jnp.float32),
            pltpu.SemaphoreType.DMA,
        ],
    )(_sc_deg_body)
    conv_kernel = functools.partial(
        pl.kernel,
        out_type=jax.ShapeDtypeStruct((NC, ACCROWS, CH), jnp.float32),
        mesh=mesh,
        scratch_types=[
            pltpu.VMEM_SHARED((ACCROWS, CH), jnp.float32),
            pltpu.VMEM((CPT, 2 * CHUNK), jnp.int32),
            pltpu.VMEM((3, CHUNK, CH), jnp.float32),
            pltpu.SemaphoreType.DMA,
            pltpu.SemaphoreType.DMA,
            pltpu.SemaphoreType.DMA,
            pltpu.SemaphoreType.DMA,
            pltpu.SemaphoreType.DMA,
            pltpu.SemaphoreType.DMA,
        ],
    )(_sc_conv_body)
    return deg_kernel, conv_kernel


def kernel(x, edge_index, W1, b1, Wmu, bmu, Wls, bls):
    src = edge_index[0].astype(jnp.int32)
    dst = edge_index[1].astype(jnp.int32)
    npad = EPAD - E
    ar = jnp.arange(npad, dtype=jnp.int32)
    src_p = jnp.concatenate([src, ar % N])
    dst_p = jnp.concatenate([dst, N + ar % (ACCROWS - N)])
    edgec = jnp.concatenate(
        [src_p.reshape(NCHUNK, CHUNK), dst_p.reshape(NCHUNK, CHUNK)], axis=1)
    dstc_deg = dst_p.reshape(DNCHUNK, DCHUNK)
    zeros128 = jnp.zeros((ACCROWS, CH), jnp.float32)
    zeros8 = jnp.zeros((ACCROWS, DW), jnp.float32)
    ones8 = jnp.ones((DCHUNK, DW), jnp.float32)

    deg_kernel, conv_kernel = _make_sc_kernels()

    degp = deg_kernel(dstc_deg, zeros8, ones8)

    m1 = pl.pallas_call(
        _tc_m1_body,
        out_shape=jax.ShapeDtypeStruct((N, CH), jnp.float32),
    )(x, W1, degp)

    p1 = conv_kernel(m1, edgec, zeros128)

    wcat = jnp.concatenate([Wmu, Wls], axis=1)
    m2 = pl.pallas_call(
        _tc_m2_body,
        out_shape=jax.ShapeDtypeStruct((N, CH), jnp.float32),
    )(p1, degp, b1, wcat)

    p2 = conv_kernel(m2, edgec, zeros128)

    mu, logstd = pl.pallas_call(
        _tc_out_body,
        out_shape=(jax.ShapeDtypeStruct((N, HALF), jnp.float32),
                   jax.ShapeDtypeStruct((N, HALF), jnp.float32)),
    )(p2, degp, bmu, bls)
    return (mu, logstd)


# speedup vs baseline: 1.1612x; 1.0042x over previous
import functools

import jax
import jax.numpy as jnp
from jax import lax
from jax.experimental import pallas as pl
from jax.experimental.pallas import tpu as pltpu
from jax.experimental.pallas import tpu_sc as plsc

N = 10000
E = 320000
CH = 128
HALF = 64
DW = 8

NC = 2
NS = 16

CHUNK = 64
CPT = 160
NCHUNK = NC * NS * CPT
EPAD = NCHUNK * CHUNK

DCHUNK = 128
DNCHUNK = EPAD // DCHUNK
DCPT = DNCHUNK // (NC * NS)

ACCROWS = 10240
RPT = ACCROWS // NS
LASTM = N - (NS - 1) * RPT


def _sc_deg_body(dstc, zeros8, ones8, out, acc, dst_v, ones_v, dsem):
    c = lax.axis_index("c")
    s = lax.axis_index("s")
    r0 = s * RPT
    pltpu.sync_copy(zeros8.at[pl.ds(r0, RPT)], acc.at[pl.ds(r0, RPT)])
    pltpu.sync_copy(ones8, ones_v)
    base = c * (DNCHUNK // NC) + s * DCPT
    pltpu.sync_copy(dstc.at[pl.ds(base, DCPT)], dst_v)
    plsc.subcore_barrier()

    def body(k, carry):
        pltpu.async_copy(ones_v, acc.at[dst_v.at[k]], dsem, add=True)

        @pl.when(k >= 8)
        def _():
            pltpu.make_async_copy(ones_v, acc.at[pl.ds(0, DCHUNK)],
                                  dsem).wait()

        return carry

    lax.fori_loop(0, DCPT, body, 0)
    for _ in range(8):
        pltpu.make_async_copy(ones_v, acc.at[pl.ds(0, DCHUNK)], dsem).wait()
    plsc.subcore_barrier()
    pltpu.sync_copy(acc.at[pl.ds(r0, RPT)], out.at[c].at[pl.ds(r0, RPT)])


def _sc_conv_body(m, edgec, zeros, out, acc, idx_v, bufs,
                  gsem0, gsem1, gsem2, ssem0, ssem1, ssem2):
    c = lax.axis_index("c")
    s = lax.axis_index("s")
    r0 = s * RPT

    gsems = (gsem0, gsem1, gsem2)
    ssems = (ssem0, ssem1, ssem2)

    def start_gather(g, r):
        pltpu.async_copy(m.at[idx_v.at[g, pl.ds(0, CHUNK)]], bufs.at[r],
                         gsems[r])

    def wait_gather(r):
        pltpu.make_async_copy(m.at[pl.ds(0, CHUNK)], bufs.at[r],
                              gsems[r]).wait()

    def start_scatter(g, r):
        pltpu.async_copy(bufs.at[r], acc.at[idx_v.at[g, pl.ds(CHUNK, CHUNK)]],
                         ssems[r], add=True)

    def wait_scatter(r):
        pltpu.make_async_copy(bufs.at[r], acc.at[pl.ds(0, CHUNK)],
                              ssems[r]).wait()

    base = c * (NCHUNK // NC) + s * CPT
    pltpu.sync_copy(edgec.at[pl.ds(base, CPT)], idx_v)

    start_gather(0, 0)
    start_gather(1, 1)

    @pl.when(jnp.logical_and(c == 0, s < NS - 1))
    def _():
        pltpu.sync_copy(m.at[pl.ds(r0, RPT)], acc.at[pl.ds(r0, RPT)])

    @pl.when(jnp.logical_and(c == 0, s == NS - 1))
    def _():
        pltpu.sync_copy(m.at[pl.ds((NS - 1) * RPT, LASTM)],
                        acc.at[pl.ds((NS - 1) * RPT, LASTM)])
        pltpu.sync_copy(zeros.at[pl.ds(N, ACCROWS - N)],
                        acc.at[pl.ds(N, ACCROWS - N)])

    @pl.when(c == 1)
    def _():
        pltpu.sync_copy(zeros.at[pl.ds(r0, RPT)], acc.at[pl.ds(r0, RPT)])

    plsc.subcore_barrier()

    def body(gg, carry):
        g0 = gg * 3
        for r in (0, 1, 2):
            g = g0 + r
            r2 = (r + 2) % 3
            wait_gather(r)
            start_scatter(g, r)

            @pl.when(g + 2 < CPT)
            def _():
                @pl.when(g >= 1)
                def _():
                    wait_scatter(r2)

                start_gather(g + 2, r2)

        return carry

    lax.fori_loop(0, CPT // 3, body, 0)
    wait_gather(0)
    start_scatter(CPT - 1, 0)
    wait_scatter(2)
    wait_scatter(1)
    wait_scatter(0)
    plsc.subcore_barrier()
    pltpu.sync_copy(acc.at[pl.ds(r0, RPT)], out.at[c].at[pl.ds(r0, RPT)])


def _dis_from_degp(degp_ref):
    deg = degp_ref[0, :, 0:1] + degp_ref[1, :, 0:1] + 1.0
    return lax.rsqrt(deg)[:N, :]


def _tc_m1_body(x_ref, w1_ref, degp_ref, out_ref):
    dis = _dis_from_degp(degp_ref)
    h = jnp.dot(x_ref[...], w1_ref[...], preferred_element_type=jnp.float32)
    out_ref[...] = h * dis


def _tc_m2_body(p_ref, degp_ref, b1_ref, wcat_ref, out_ref):
    dis = _dis_from_degp(degp_ref)
    a = p_ref[0, :N, :] + p_ref[1, :N, :]
    h = jnp.maximum(dis * a + b1_ref[...][None, :], 0.0)
    out_ref[...] = jnp.dot(h, wcat_ref[...],
                           preferred_element_type=jnp.float32) * dis


def _tc_out_body(q_ref, degp_ref, bmu_ref, bls_ref, mu_ref, ls_ref):
    dis = _dis_from_degp(degp_ref)
    o = dis * (q_ref[0, :N, :] + q_ref[1, :N, :])
    mu_ref[...] = o[:, :HALF] + bmu_ref[...][None, :]
    ls_ref[...] = o[:, HALF:] + bls_ref[...][None, :]


def _make_sc_kernels():
    mesh = plsc.VectorSubcoreMesh(core_axis_name="c", subcore_axis_name="s")
    deg_kernel = functools.partial(
        pl.kernel,
        out_type=jax.ShapeDtypeStruct((NC, ACCROWS, DW), jnp.float32),
        mesh=mesh,
        compiler_params=pltpu.CompilerParams(use_tc_tiling_on_sc=False),
        scratch_types=[
            pltpu.VMEM_SHARED((ACCROWS, DW), jnp.float32),
            pltpu.VMEM((DCPT, DCHUNK), jnp.int32),
            pltpu.VMEM((DCHUNK, DW), jnp.float32),
            pltpu.SemaphoreType.DMA,
        ],
    )(_sc_deg_body)
    conv_kernel = functools.partial(
        pl.kernel,
        out_type=jax.ShapeDtypeStruct((NC, ACCROWS, CH), jnp.float32),
        mesh=mesh,
        scratch_types=[
            pltpu.VMEM_SHARED((ACCROWS, CH), jnp.float32),
            pltpu.VMEM((CPT, 2 * CHUNK), jnp.int32),
            pltpu.VMEM((3, CHUNK, CH), jnp.float32),
            pltpu.SemaphoreType.DMA,
            pltpu.SemaphoreType.DMA,
            pltpu.SemaphoreType.DMA,
            pltpu.SemaphoreType.DMA,
            pltpu.SemaphoreType.DMA,
            pltpu.SemaphoreType.DMA,
        ],
    )(_sc_conv_body)
    return deg_kernel, conv_kernel


def kernel(x, edge_index, W1, b1, Wmu, bmu, Wls, bls):
    src = edge_index[0].astype(jnp.int32)
    dst = edge_index[1].astype(jnp.int32)
    npad = EPAD - E
    ar = jnp.arange(npad, dtype=jnp.int32)
    src_p = jnp.concatenate([src, ar % N])
    dst_p = jnp.concatenate([dst, N + ar % (ACCROWS - N)])
    edgec = jnp.concatenate(
        [src_p.reshape(NCHUNK, CHUNK), dst_p.reshape(NCHUNK, CHUNK)], axis=1)
    dstc_deg = dst_p.reshape(DNCHUNK, DCHUNK)
    zeros128 = jnp.zeros((ACCROWS, CH), jnp.float32)
    zeros8 = jnp.zeros((ACCROWS, DW), jnp.float32)
    ones8 = jnp.ones((DCHUNK, DW), jnp.float32)

    deg_kernel, conv_kernel = _make_sc_kernels()

    degp = deg_kernel(dstc_deg, zeros8, ones8)

    m1 = pl.pallas_call(
        _tc_m1_body,
        out_shape=jax.ShapeDtypeStruct((N, CH), jnp.float32),
    )(x, W1, degp)

    p1 = conv_kernel(m1, edgec, zeros128)

    wcat = jnp.concatenate([Wmu, Wls], axis=1)
    m2 = pl.pallas_call(
        _tc_m2_body,
        out_shape=jax.ShapeDtypeStruct((N, CH), jnp.float32),
    )(p1, degp, b1, wcat)

    p2 = conv_kernel(m2, edgec, zeros128)

    mu, logstd = pl.pallas_call(
        _tc_out_body,
        out_shape=(jax.ShapeDtypeStruct((N, HALF), jnp.float32),
                   jax.ShapeDtypeStruct((N, HALF), jnp.float32)),
    )(p2, degp, bmu, bls)
    return (mu, logstd)


# speedup vs baseline: 1.2045x; 1.0373x over previous
import functools

import jax
import jax.numpy as jnp
import numpy as np
from jax import lax
from jax.experimental import pallas as pl
from jax.experimental.pallas import tpu as pltpu
from jax.experimental.pallas import tpu_sc as plsc

N = 10000
E = 320000
CH = 128
HALF = 64
DW = 8

NC = 2
NS = 16

CHUNK = 64
CPT = 160
NCHUNK = NC * NS * CPT
EPAD = NCHUNK * CHUNK

DCHUNK = 128
DNCHUNK = EPAD // DCHUNK
DCPT = DNCHUNK // (NC * NS)

ACCROWS = 10240
RPT = ACCROWS // NS
LASTM = N - (NS - 1) * RPT


def _sc_deg_body(dstc, zeros8, ones8, out, acc, dst_v, ones_v, dsem):
    c = lax.axis_index("c")
    s = lax.axis_index("s")
    r0 = s * RPT
    pltpu.sync_copy(zeros8.at[pl.ds(r0, RPT)], acc.at[pl.ds(r0, RPT)])
    pltpu.sync_copy(ones8, ones_v)
    base = c * (DNCHUNK // NC) + s * DCPT
    pltpu.sync_copy(dstc.at[pl.ds(base, DCPT)], dst_v)
    plsc.subcore_barrier()

    def body(k, carry):
        pltpu.async_copy(ones_v, acc.at[dst_v.at[k]], dsem, add=True)

        @pl.when(k >= 8)
        def _():
            pltpu.make_async_copy(ones_v, acc.at[pl.ds(0, DCHUNK)],
                                  dsem).wait()

        return carry

    lax.fori_loop(0, DCPT, body, 0)
    for _ in range(8):
        pltpu.make_async_copy(ones_v, acc.at[pl.ds(0, DCHUNK)], dsem).wait()
    plsc.subcore_barrier()
    pltpu.sync_copy(acc.at[pl.ds(r0, RPT)], out.at[c].at[pl.ds(r0, RPT)])


def _sc_conv_body(m, edgec, zeros, out, acc, idx_v, bufs,
                  gsem0, gsem1, gsem2, ssem0, ssem1, ssem2):
    c = lax.axis_index("c")
    s = lax.axis_index("s")
    r0 = s * RPT

    gsems = (gsem0, gsem1, gsem2)
    ssems = (ssem0, ssem1, ssem2)

    def start_gather(g, r):
        pltpu.async_copy(m.at[idx_v.at[g, pl.ds(0, CHUNK)]], bufs.at[r],
                         gsems[r])

    def wait_gather(r):
        pltpu.make_async_copy(m.at[pl.ds(0, CHUNK)], bufs.at[r],
                              gsems[r]).wait()

    def start_scatter(g, r):
        pltpu.async_copy(bufs.at[r], acc.at[idx_v.at[g, pl.ds(CHUNK, CHUNK)]],
                         ssems[r], add=True)

    def wait_scatter(r):
        pltpu.make_async_copy(bufs.at[r], acc.at[pl.ds(0, CHUNK)],
                              ssems[r]).wait()

    base = c * (NCHUNK // NC) + s * CPT
    pltpu.sync_copy(edgec.at[pl.ds(base, CPT)], idx_v)

    start_gather(0, 0)
    start_gather(1, 1)

    @pl.when(jnp.logical_and(c == 0, s < NS - 1))
    def _():
        pltpu.sync_copy(m.at[pl.ds(r0, RPT)], acc.at[pl.ds(r0, RPT)])

    @pl.when(jnp.logical_and(c == 0, s == NS - 1))
    def _():
        pltpu.sync_copy(m.at[pl.ds((NS - 1) * RPT, LASTM)],
                        acc.at[pl.ds((NS - 1) * RPT, LASTM)])
        pltpu.sync_copy(zeros.at[pl.ds(N, ACCROWS - N)],
                        acc.at[pl.ds(N, ACCROWS - N)])

    @pl.when(c == 1)
    def _():
        pltpu.sync_copy(zeros.at[pl.ds(r0, RPT)], acc.at[pl.ds(r0, RPT)])

    plsc.subcore_barrier()

    def body(gg, carry):
        g0 = gg * 3
        for r in (0, 1, 2):
            g = g0 + r
            r2 = (r + 2) % 3
            wait_gather(r)
            start_scatter(g, r)

            @pl.when(g + 2 < CPT)
            def _():
                @pl.when(g >= 1)
                def _():
                    wait_scatter(r2)

                start_gather(g + 2, r2)

        return carry

    lax.fori_loop(0, CPT // 3, body, 0)
    wait_gather(0)
    start_scatter(CPT - 1, 0)
    wait_scatter(2)
    wait_scatter(1)
    wait_scatter(0)
    plsc.subcore_barrier()
    pltpu.sync_copy(acc.at[pl.ds(r0, RPT)], out.at[c].at[pl.ds(r0, RPT)])


def _tc_prep_body(e_ref, pads_ref, ec_ref, dd_ref):
    e3 = e_ref[...].reshape(2, E // DCHUNK, DCHUNK)
    src2 = jnp.concatenate([e3[0], pads_ref[0]], 0)
    dst2 = jnp.concatenate([e3[1], pads_ref[1]], 0)
    src3 = src2.reshape(DNCHUNK, 2, CHUNK)
    dst3 = dst2.reshape(DNCHUNK, 2, CHUNK)
    cat = jnp.concatenate([src3, dst3], 2)
    ec_ref[...] = cat.reshape(NCHUNK, 2 * CHUNK)
    dd_ref[...] = dst2


def _dis_from_degp(degp_ref):
    deg = degp_ref[0, :, 0:1] + degp_ref[1, :, 0:1] + 1.0
    return lax.rsqrt(deg)[:N, :]


def _tc_m1_body(x_ref, w1_ref, degp_ref, out_ref):
    dis = _dis_from_degp(degp_ref)
    h = jnp.dot(x_ref[...], w1_ref[...], preferred_element_type=jnp.float32)
    out_ref[...] = h * dis


def _tc_m2_body(p_ref, degp_ref, b1_ref, wcat_ref, out_ref):
    dis = _dis_from_degp(degp_ref)
    a = p_ref[0, :N, :] + p_ref[1, :N, :]
    h = jnp.maximum(dis * a + b1_ref[...][None, :], 0.0)
    out_ref[...] = jnp.dot(h, wcat_ref[...],
                           preferred_element_type=jnp.float32) * dis


def _tc_out_body(q_ref, degp_ref, bmu_ref, bls_ref, mu_ref, ls_ref):
    dis = _dis_from_degp(degp_ref)
    o = dis * (q_ref[0, :N, :] + q_ref[1, :N, :])
    mu_ref[...] = o[:, :HALF] + bmu_ref[...][None, :]
    ls_ref[...] = o[:, HALF:] + bls_ref[...][None, :]


def _make_sc_kernels():
    mesh = plsc.VectorSubcoreMesh(core_axis_name="c", subcore_axis_name="s")
    deg_kernel = functools.partial(
        pl.kernel,
        out_type=jax.ShapeDtypeStruct((NC, ACCROWS, DW), jnp.float32),
        mesh=mesh,
        compiler_params=pltpu.CompilerParams(use_tc_tiling_on_sc=False),
        scratch_types=[
            pltpu.VMEM_SHARED((ACCROWS, DW), jnp.float32),
            pltpu.VMEM((DCPT, DCHUNK), jnp.int32),
            pltpu.VMEM((DCHUNK, DW), jnp.float32),
            pltpu.SemaphoreType.DMA,
        ],
    )(_sc_deg_body)
    conv_kernel = functools.partial(
        pl.kernel,
        out_type=jax.ShapeDtypeStruct((NC, ACCROWS, CH), jnp.float32),
        mesh=mesh,
        scratch_types=[
            pltpu.VMEM_SHARED((ACCROWS, CH), jnp.float32),
            pltpu.VMEM((CPT, 2 * CHUNK), jnp.int32),
            pltpu.VMEM((3, CHUNK, CH), jnp.float32),
            pltpu.SemaphoreType.DMA,
            pltpu.SemaphoreType.DMA,
            pltpu.SemaphoreType.DMA,
            pltpu.SemaphoreType.DMA,
            pltpu.SemaphoreType.DMA,
            pltpu.SemaphoreType.DMA,
        ],
    )(_sc_conv_body)
    return deg_kernel, conv_kernel


def kernel(x, edge_index, W1, b1, Wmu, bmu, Wls, bls):
    npad = EPAD - E
    ar = np.arange(npad, dtype=np.int32)
    pads = jnp.asarray(
        np.stack([(ar % N).reshape(npad // DCHUNK, DCHUNK),
                  (N + ar % (ACCROWS - N)).reshape(npad // DCHUNK, DCHUNK)]))
    zeros128 = jnp.zeros((ACCROWS, CH), jnp.float32)
    zeros8 = jnp.zeros((ACCROWS, DW), jnp.float32)
    ones8 = jnp.ones((DCHUNK, DW), jnp.float32)

    deg_kernel, conv_kernel = _make_sc_kernels()

    edgec, dstc_deg = pl.pallas_call(
        _tc_prep_body,
        out_shape=(jax.ShapeDtypeStruct((NCHUNK, 2 * CHUNK), jnp.int32),
                   jax.ShapeDtypeStruct((DNCHUNK, DCHUNK), jnp.int32)),
    )(edge_index, pads)

    degp = deg_kernel(dstc_deg, zeros8, ones8)

    m1 = pl.pallas_call(
        _tc_m1_body,
        out_shape=jax.ShapeDtypeStruct((N, CH), jnp.float32),
    )(x, W1, degp)

    p1 = conv_kernel(m1, edgec, zeros128)

    wcat = jnp.concatenate([Wmu, Wls], axis=1)
    m2 = pl.pallas_call(
        _tc_m2_body,
        out_shape=jax.ShapeDtypeStruct((N, CH), jnp.float32),
    )(p1, degp, b1, wcat)

    p2 = conv_kernel(m2, edgec, zeros128)

    mu, logstd = pl.pallas_call(
        _tc_out_body,
        out_shape=(jax.ShapeDtypeStruct((N, HALF), jnp.float32),
                   jax.ShapeDtypeStruct((N, HALF), jnp.float32)),
    )(p2, degp, bmu, bls)
    return (mu, logstd)
